# Initial kernel scaffold; baseline (speedup 1.0000x reference)
import jax
import jax.numpy as jnp
from jax.experimental import pallas as pl


def kernel(texc, hash_table, W1, W2, W3):
    raise NotImplementedError("write your pallas kernel here")


# speedup vs baseline: 51.6434x; 51.6434x over previous
import functools

import jax
import jax.numpy as jnp
import numpy as np
from jax import lax
from jax.experimental import pallas as pl
from jax.experimental.pallas import tpu as pltpu
from jax.experimental.pallas import tpu_sc as plsc

NUM_LEVELS = 16
F = 2
T = 2 ** 19
BASE_RES = 16.0
PER_LEVEL_SCALE = float(np.exp(np.log(4096.0 / 16.0) / (NUM_LEVELS - 1)))
RES = [float(np.floor(BASE_RES * (PER_LEVEL_SCALE ** l))) for l in range(NUM_LEVELS)]
P1_2 = np.uint32((2654435761 * 2) & 0xFFFFFFFF)
P2_2 = np.uint32((805459861 * 2) & 0xFFFFFFFF)
MASK2 = np.uint32((T - 1) << 1)
CHANNELS = 3
INTERNAL = 32
N_POINTS = 1048576

NC = 2
NS = 16
NW = NC * NS
NPW = N_POINTS // NW
C = 1024
NCHUNK = NPW // C
G = C // 16


def _worker_id():
    return lax.axis_index("s") * NC + lax.axis_index("c")


def _fire(table_hbm, idxs, f0buf, f1buf, sem):
    for c in range(8):
        dst = pl.ds(c * C, C)
        pltpu.async_copy(table_hbm.at[idxs.at[c, 0]], f0buf.at[dst], sem)
        pltpu.async_copy(table_hbm.at[idxs.at[c + 8, 0]], f1buf.at[dst], sem)


def _drain(table_hbm, idxs, f0buf, f1buf, sem):
    for c in range(8):
        dst = pl.ds(c * C, C)
        pltpu.make_async_copy(table_hbm.at[idxs.at[c, 0]], f0buf.at[dst], sem).wait()
        pltpu.make_async_copy(table_hbm.at[idxs.at[c + 8, 0]], f1buf.at[dst], sem).wait()


def _encode_body(x_hbm, y_hbm, z_hbm, table_hbm, penc_hbm,
                 xn, yn, zn,
                 wbufs0, wbufs1, idxs0, idxs1,
                 f0a, f1a, f0b, f1b, featsT, semA, semB):
    wid = _worker_id()
    wsets = (wbufs0, wbufs1)
    isets = (idxs0, idxs1)
    fsets = ((f0a, f1a, semA), (f0b, f1b, semB))

    def hash_pass(l, sel):
        res = RES[l]
        off2 = np.uint32((l * T) << 1)
        wb = wsets[sel]
        ib = isets[sel]

        def hash_g(g, c2):
            s = g * 16
            px = xn[pl.ds(s, 16)] * res
            py = yn[pl.ds(s, 16)] * res
            pz = zn[pl.ds(s, 16)] * res
            ix = px.astype(jnp.int32)
            iy = py.astype(jnp.int32)
            iz = pz.astype(jnp.int32)
            wb[0, pl.ds(s, 16)] = px - ix.astype(jnp.float32)
            wb[1, pl.ds(s, 16)] = py - iy.astype(jnp.float32)
            wb[2, pl.ds(s, 16)] = pz - iz.astype(jnp.float32)
            ux2 = ix.astype(jnp.uint32) * np.uint32(2)
            uy2 = iy.astype(jnp.uint32) * P1_2
            uz2 = iz.astype(jnp.uint32) * P2_2
            hx = (ux2, ux2 + np.uint32(2))
            hy = (uy2, uy2 + P1_2)
            hz = (uz2, uz2 + P2_2)
            c = 0
            for i in range(2):
                for j in range(2):
                    for k in range(2):
                        i0 = (((hx[i] ^ hy[j] ^ hz[k]) & MASK2) | off2).astype(jnp.int32)
                        ib[c, 0, pl.ds(s, 16)] = i0
                        ib[c + 8, 0, pl.ds(s, 16)] = i0 + 1
                        c += 1
            return c2

        lax.fori_loop(0, G, hash_g, 0)

    def interp_pass(l, sel):
        wb = wsets[sel]
        f0buf, f1buf, _ = fsets[sel]

        def interp_g(g, c2):
            s = g * 16
            wxv = wb[0, pl.ds(s, 16)]
            wyv = wb[1, pl.ds(s, 16)]
            wzv = wb[2, pl.ds(s, 16)]
            outs = []
            for fbuf in (f0buf, f1buf):
                fc = [fbuf[pl.ds(c * C + s, 16)] for c in range(8)]
                t00 = fc[0] + wzv * (fc[1] - fc[0])
                t01 = fc[2] + wzv * (fc[3] - fc[2])
                t10 = fc[4] + wzv * (fc[5] - fc[4])
                t11 = fc[6] + wzv * (fc[7] - fc[6])
                u0 = t00 + wyv * (t01 - t00)
                u1 = t10 + wyv * (t11 - t10)
                outs.append(u0 + wxv * (u1 - u0))
            featsT[2 * l, pl.ds(s, 16)] = outs[0]
            featsT[2 * l + 1, pl.ds(s, 16)] = outs[1]
            return c2

        lax.fori_loop(0, G, interp_g, 0)

    def chunk_body(ci, carry):
        base = wid * NPW + ci * C
        pltpu.sync_copy(x_hbm.at[pl.ds(base, C)], xn)
        pltpu.sync_copy(y_hbm.at[pl.ds(base, C)], yn)
        pltpu.sync_copy(z_hbm.at[pl.ds(base, C)], zn)

        def norm_g(g, c2):
            s = g * 16
            for ref in (xn, yn, zn):
                v = (ref[pl.ds(s, 16)] + 1.0) * 0.5
                v = jnp.minimum(jnp.maximum(v, 0.0), 1.0)
                ref[pl.ds(s, 16)] = v
            return c2

        lax.fori_loop(0, G, norm_g, 0)

        hash_pass(0, 0)
        _fire(table_hbm, isets[0], fsets[0][0], fsets[0][1], fsets[0][2])
        for l in range(NUM_LEVELS):
            cur = l % 2
            nxt = (l + 1) % 2
            if l + 1 < NUM_LEVELS:
                hash_pass(l + 1, nxt)
                _fire(table_hbm, isets[nxt], fsets[nxt][0], fsets[nxt][1], fsets[nxt][2])
            _drain(table_hbm, isets[cur], fsets[cur][0], fsets[cur][1], fsets[cur][2])
            interp_pass(l, cur)

        pltpu.sync_copy(featsT, penc_hbm.at[:, pl.ds(base, C)])
        return carry

    lax.fori_loop(0, NCHUNK, chunk_body, 0)


def _encode(xs, ys, zs, table_flat):
    mesh = plsc.VectorSubcoreMesh(core_axis_name="c", subcore_axis_name="s",
                                  num_cores=NC, num_subcores=NS)
    return pl.kernel(
        _encode_body,
        out_type=jax.ShapeDtypeStruct((NUM_LEVELS * F, N_POINTS), jnp.float32),
        mesh=mesh,
        scratch_types=[
            pltpu.VMEM((C,), jnp.float32),
            pltpu.VMEM((C,), jnp.float32),
            pltpu.VMEM((C,), jnp.float32),
            pltpu.VMEM((3, C), jnp.float32),
            pltpu.VMEM((3, C), jnp.float32),
            pltpu.VMEM((16, 1, C), jnp.int32),
            pltpu.VMEM((16, 1, C), jnp.int32),
            pltpu.VMEM((8 * C,), jnp.float32),
            pltpu.VMEM((8 * C,), jnp.float32),
            pltpu.VMEM((8 * C,), jnp.float32),
            pltpu.VMEM((8 * C,), jnp.float32),
            pltpu.VMEM((NUM_LEVELS * F, C), jnp.float32),
            pltpu.SemaphoreType.DMA,
            pltpu.SemaphoreType.DMA,
        ],
    )(xs, ys, zs, table_flat)


NB = 8192


def _mlp_body(x_ref, w1_ref, w2_ref, w3_ref, o_ref):
    x = x_ref[...]
    h = jnp.maximum(jnp.dot(w1_ref[...], x, preferred_element_type=jnp.float32), 0.0)
    h = jnp.maximum(jnp.dot(w2_ref[...], h, preferred_element_type=jnp.float32), 0.0)
    z = jnp.dot(w3_ref[...], h, preferred_element_type=jnp.float32)
    o_ref[...] = jax.nn.sigmoid(z)


def _mlp(pencT, W1, W2, W3):
    grid = (N_POINTS // NB,)
    return pl.pallas_call(
        _mlp_body,
        grid=grid,
        in_specs=[
            pl.BlockSpec((NUM_LEVELS * F, NB), lambda i: (0, i)),
            pl.BlockSpec((INTERNAL, NUM_LEVELS * F), lambda i: (0, 0)),
            pl.BlockSpec((INTERNAL, INTERNAL), lambda i: (0, 0)),
            pl.BlockSpec((CHANNELS, INTERNAL), lambda i: (0, 0)),
        ],
        out_specs=pl.BlockSpec((CHANNELS, NB), lambda i: (0, i)),
        out_shape=jax.ShapeDtypeStruct((CHANNELS, N_POINTS), jnp.float32),
    )(pencT, W1, W2, W3)


def kernel(texc, hash_table, W1, W2, W3):
    lead_shape = texc.shape[:-1]
    texc2 = texc.reshape(-1, 3)
    table_flat = hash_table.reshape(NUM_LEVELS * T * F)
    pencT = _encode(texc2[:, 0], texc2[:, 1], texc2[:, 2], table_flat)
    outT = _mlp(pencT, W1, W2, W3)
    return outT.T.reshape(lead_shape + (CHANNELS,))


# speedup vs baseline: 371.5037x; 7.1936x over previous
import functools

import jax
import jax.numpy as jnp
import numpy as np
from jax import lax
from jax.experimental import pallas as pl
from jax.experimental.pallas import tpu as pltpu
from jax.experimental.pallas import tpu_sc as plsc

NUM_LEVELS = 16
F = 2
T = 2 ** 19
BASE_RES = 16.0
PER_LEVEL_SCALE = float(np.exp(np.log(4096.0 / 16.0) / (NUM_LEVELS - 1)))
RES = [float(np.floor(BASE_RES * (PER_LEVEL_SCALE ** l))) for l in range(NUM_LEVELS)]
P1 = np.uint32(2654435761)
P2 = np.uint32(805459861)
MASK = np.uint32(T - 1)
CHANNELS = 3
INTERNAL = 32
N_POINTS = 1048576

NC = 2
NS = 16
NW = NC * NS
NPW = N_POINTS // NW
C = 2048
NCHUNK = NPW // C
G = C // 16
TS = T // NS


def _worker_id():
    return lax.axis_index("s") * NC + lax.axis_index("c")


def _subcore_id():
    return lax.axis_index("s")


def _fire(spm, idxs, fbuf, sem):
    pltpu.async_copy(spm.at[idxs.at[0, 0]], fbuf, sem)


def _drain(spm, idxs, fbuf, sem):
    pltpu.make_async_copy(spm.at[idxs.at[0, 0]], fbuf, sem).wait()


def _stage(table_hbm, spm, l, sid):
    pltpu.sync_copy(table_hbm.at[pl.ds(l * T + sid * TS, TS)],
                    spm.at[pl.ds(sid * TS, TS)])


def _barrier():
    plsc.subcore_barrier()


def _encode_body(x_hbm, y_hbm, z_hbm, table_hbm, res_hbm, penc_hbm,
                 xn, yn, zn, resv,
                 wbufs0, wbufs1, idxs0, idxs1,
                 fba, fbb, fpair, spm0, semA, semB):
    wid = _worker_id()
    sid = _subcore_id()
    pltpu.sync_copy(res_hbm, resv)
    wsets = (wbufs0, wbufs1)
    isets = (idxs0, idxs1)
    fsets = ((fba, semA), (fbb, semB))

    def load_coords(ci):
        base = wid * NPW + ci * C
        pltpu.sync_copy(x_hbm.at[pl.ds(base, C)], xn)
        pltpu.sync_copy(y_hbm.at[pl.ds(base, C)], yn)
        pltpu.sync_copy(z_hbm.at[pl.ds(base, C)], zn)

    def hash_pass(res, sel):
        wb = wsets[sel]
        ib = isets[sel]

        def hash_g(g, c2, res=res):
            s = g * 16
            px = xn[pl.ds(s, 16)] * res
            py = yn[pl.ds(s, 16)] * res
            pz = zn[pl.ds(s, 16)] * res
            ix = px.astype(jnp.int32)
            iy = py.astype(jnp.int32)
            iz = pz.astype(jnp.int32)
            wb[0, pl.ds(s, 16)] = px - ix.astype(jnp.float32)
            wb[1, pl.ds(s, 16)] = py - iy.astype(jnp.float32)
            wb[2, pl.ds(s, 16)] = pz - iz.astype(jnp.float32)
            ux = ix.astype(jnp.uint32)
            uy = iy.astype(jnp.uint32) * P1
            uz = iz.astype(jnp.uint32) * P2
            hx = (ux, ux + np.uint32(1))
            hy = (uy, uy + P1)
            hz = (uz, uz + P2)
            c = 0
            for i in range(2):
                for j in range(2):
                    for k in range(2):
                        i0 = ((hx[i] ^ hy[j] ^ hz[k]) & MASK).astype(jnp.int32)
                        ib[0, 0, pl.ds(c * C + s, 16)] = i0
                        c += 1
            return c2

        lax.fori_loop(0, G, hash_g, 0)

    def interp_pass(sel):
        wb = wsets[sel]
        fbuf, _ = fsets[sel]

        def interp_g(g, c2):
            s = g * 16
            wxv = wb[0, pl.ds(s, 16)]
            wyv = wb[1, pl.ds(s, 16)]
            wzv = wb[2, pl.ds(s, 16)]
            f0c, f1c = [], []
            for c in range(8):
                p = lax.bitcast_convert_type(fbuf[pl.ds(c * C + s, 16)], jnp.uint32)
                f0c.append(lax.bitcast_convert_type(p << 16, jnp.float32))
                f1c.append(lax.bitcast_convert_type(p & np.uint32(0xFFFF0000),
                                                    jnp.float32))
            outs = []
            for fc in (f0c, f1c):
                t00 = fc[0] + wzv * (fc[1] - fc[0])
                t01 = fc[2] + wzv * (fc[3] - fc[2])
                t10 = fc[4] + wzv * (fc[5] - fc[4])
                t11 = fc[6] + wzv * (fc[7] - fc[6])
                u0 = t00 + wyv * (t01 - t00)
                u1 = t10 + wyv * (t11 - t10)
                outs.append(u0 + wxv * (u1 - u0))
            fpair[0, pl.ds(s, 16)] = outs[0]
            fpair[1, pl.ds(s, 16)] = outs[1]
            return c2

        lax.fori_loop(0, G, interp_g, 0)

    def level_body(l, carry):
        spm = spm0
        _stage(table_hbm, spm, l, sid)
        _barrier()
        res = resv[l, pl.ds(0, 16)]

        def finish_chunk(ci, sel):
            _drain(spm, isets[sel], fsets[sel][0], fsets[sel][1])
            interp_pass(sel)
            base = wid * NPW + ci * C
            pltpu.sync_copy(fpair, penc_hbm.at[pl.ds(2 * l, 2), pl.ds(base, C)])

        def start_chunk(ci, sel):
            load_coords(ci)
            hash_pass(res, sel)
            _fire(spm, isets[sel], fsets[sel][0], fsets[sel][1])

        start_chunk(0, 0)

        def chunk_pair(i, c2):
            ci0 = 2 * i
            ci1 = ci0 + 1
            start_chunk(ci1, 1)
            finish_chunk(ci0, 0)

            @pl.when(ci1 + 1 < NCHUNK)
            def _():
                start_chunk(ci1 + 1, 0)

            finish_chunk(ci1, 1)
            return c2

        lax.fori_loop(0, NCHUNK // 2, chunk_pair, 0)
        _barrier()
        return carry

    lax.fori_loop(0, NUM_LEVELS, level_body, 0)


def _encode(xs, ys, zs, table_flat):
    mesh = plsc.VectorSubcoreMesh(core_axis_name="c", subcore_axis_name="s",
                                  num_cores=NC, num_subcores=NS)
    return pl.kernel(
        _encode_body,
        out_type=jax.ShapeDtypeStruct((NUM_LEVELS * F, N_POINTS), jnp.float32),
        mesh=mesh,
        scratch_types=[
            pltpu.VMEM((C,), jnp.float32),
            pltpu.VMEM((C,), jnp.float32),
            pltpu.VMEM((C,), jnp.float32),
            pltpu.VMEM((16, 16), jnp.float32),
            pltpu.VMEM((3, C), jnp.float32),
            pltpu.VMEM((3, C), jnp.float32),
            pltpu.VMEM((1, 1, 8 * C), jnp.int32),
            pltpu.VMEM((1, 1, 8 * C), jnp.int32),
            pltpu.VMEM((8 * C,), jnp.float32),
            pltpu.VMEM((8 * C,), jnp.float32),
            pltpu.VMEM((2, C), jnp.float32),
            pltpu.VMEM_SHARED((T,), jnp.float32),
            pltpu.SemaphoreType.DMA,
            pltpu.SemaphoreType.DMA,
        ],
    )(xs, ys, zs, table_flat,
      jnp.asarray(np.tile(np.array(RES, dtype=np.float32)[:, None], (1, 16))))


NB = 8192


def _split_body(t_ref, x_ref, y_ref, z_ref):
    t = t_ref[...]
    tn = jnp.clip((t + 1.0) * 0.5, 0.0, 1.0)
    tt = tn.T
    x_ref[...] = tt[0, :]
    y_ref[...] = tt[1, :]
    z_ref[...] = tt[2, :]


def _split(texc2):
    grid = (N_POINTS // NB,)
    out = jax.ShapeDtypeStruct((N_POINTS,), jnp.float32)
    return pl.pallas_call(
        _split_body,
        grid=grid,
        in_specs=[pl.BlockSpec((NB, 3), lambda i: (i, 0))],
        out_specs=[pl.BlockSpec((NB,), lambda i: (i,))] * 3,
        out_shape=[out, out, out],
    )(texc2)


def _mlp_body(x_ref, w1_ref, w2_ref, w3_ref, o_ref):
    x = x_ref[...]
    h = jnp.maximum(jnp.dot(w1_ref[...], x, preferred_element_type=jnp.float32), 0.0)
    h = jnp.maximum(jnp.dot(w2_ref[...], h, preferred_element_type=jnp.float32), 0.0)
    z = jnp.dot(w3_ref[...], h, preferred_element_type=jnp.float32)
    o_ref[...] = jax.nn.sigmoid(z).T


def _mlp(pencT, W1, W2, W3):
    grid = (N_POINTS // NB,)
    return pl.pallas_call(
        _mlp_body,
        grid=grid,
        in_specs=[
            pl.BlockSpec((NUM_LEVELS * F, NB), lambda i: (0, i)),
            pl.BlockSpec((INTERNAL, NUM_LEVELS * F), lambda i: (0, 0)),
            pl.BlockSpec((INTERNAL, INTERNAL), lambda i: (0, 0)),
            pl.BlockSpec((CHANNELS, INTERNAL), lambda i: (0, 0)),
        ],
        out_specs=pl.BlockSpec((NB, CHANNELS), lambda i: (i, 0)),
        out_shape=jax.ShapeDtypeStruct((N_POINTS, CHANNELS), jnp.float32),
    )(pencT, W1, W2, W3)


def kernel(texc, hash_table, W1, W2, W3):
    lead_shape = texc.shape[:-1]
    texc2 = texc.reshape(-1, 3)
    table_flat = lax.bitcast_convert_type(
        hash_table.astype(jnp.bfloat16).reshape(NUM_LEVELS * T, F),
        jnp.float32)
    xs, ys, zs = _split(texc2)
    pencT = _encode(xs, ys, zs, table_flat)
    out = _mlp(pencT, W1, W2, W3)
    return out.reshape(lead_shape + (CHANNELS,))
